# Initial kernel scaffold; baseline (speedup 1.0000x reference)
import jax
import jax.numpy as jnp
from jax.experimental import pallas as pl


def kernel(x, edge_index, edge_attr, batch, em_w1, em_b1, em_w2, em_b2, c1_lw, c1_lb, c1_w1, c1_b1, c1_w2, c1_b2, c2_lw, c2_lb, c2_w1, c2_b1, c2_w2, c2_b2, l1_w, l1_b, l2_w, l2_b, hS_w, hS_b, hP_w, hP_b, hN_w, hN_b):
    raise NotImplementedError("write your pallas kernel here")


# speedup vs baseline: 2.5543x; 2.5543x over previous
import functools

import jax
import jax.numpy as jnp
from jax import lax
from jax.experimental import pallas as pl
from jax.experimental.pallas import tpu as pltpu
from jax.experimental.pallas import tpu_sc as plsc

N = 10000
E = 320000
G = 64
D_IN = 128
H = 256

_EB = 4000


def _edge_mlp_body(ea_ref, w1_ref, b1_ref, w2_ref, b2_ref, l1w_ref, l1b_ref,
                   l2w_ref, l2b_ref, e1_ref, e2_ref):
    t = jnp.dot(ea_ref[...], w1_ref[...], preferred_element_type=jnp.float32)
    t = jnp.maximum(t + b1_ref[...], 0.0)
    ea = jnp.dot(t, w2_ref[...], preferred_element_type=jnp.float32) + b2_ref[...]
    r1 = jnp.dot(ea, l1w_ref[...], preferred_element_type=jnp.float32) + l1b_ref[...]
    r2 = jnp.dot(ea, l2w_ref[...], preferred_element_type=jnp.float32) + l2b_ref[...]
    e1_ref[...] = r1
    e2_ref[0] = r2[:, :128]
    e2_ref[1] = r2[:, 128:]


def _edge_features(edge_attr, em_w1, em_b1, em_w2, em_b2, c1_lw, c1_lb,
                   c2_lw, c2_lb):
    nblk = E // _EB
    return pl.pallas_call(
        _edge_mlp_body,
        grid=(nblk,),
        in_specs=[
            pl.BlockSpec((_EB, 16), lambda i: (i, 0)),
            pl.BlockSpec((16, H), lambda i: (0, 0)),
            pl.BlockSpec((1, H), lambda i: (0, 0)),
            pl.BlockSpec((H, H), lambda i: (0, 0)),
            pl.BlockSpec((1, H), lambda i: (0, 0)),
            pl.BlockSpec((H, D_IN), lambda i: (0, 0)),
            pl.BlockSpec((1, D_IN), lambda i: (0, 0)),
            pl.BlockSpec((H, H), lambda i: (0, 0)),
            pl.BlockSpec((1, H), lambda i: (0, 0)),
        ],
        out_specs=[
            pl.BlockSpec((_EB, 128), lambda i: (i, 0)),
            pl.BlockSpec((2, _EB, 128), lambda i: (0, i, 0)),
        ],
        out_shape=[
            jax.ShapeDtypeStruct((E, 128), jnp.float32),
            jax.ShapeDtypeStruct((2, E, 128), jnp.float32),
        ],
    )(edge_attr, em_w1, em_b1, em_w2, em_b2, c1_lw, c1_lb, c2_lw, c2_lb)


_C = 128
_NP = 10240
_RPT = _NP // 16
_RB = 64


def _make_sc_scatter(feat_split):
    D = 128
    nch = (E // _C) // 2 if not feat_split else E // _C
    kbase, kextra = nch // 16, nch % 16
    mesh = plsc.VectorSubcoreMesh(core_axis_name="c", subcore_axis_name="s")

    @functools.partial(
        pl.kernel,
        out_type=jax.ShapeDtypeStruct((2 * _NP, D), jnp.float32),
        mesh=mesh,
        scratch_types=[
            pltpu.VMEM((_C,), jnp.int32),
            pltpu.VMEM((_C,), jnp.int32),
            pltpu.VMEM((_C,), jnp.int32),
            pltpu.VMEM((_C, D), jnp.float32),
            pltpu.VMEM((_C, D), jnp.float32),
            pltpu.VMEM((_RB, D), jnp.float32),
            pltpu.VMEM_SHARED((_NP, D), jnp.float32),
            pltpu.SemaphoreType.DMA,
        ],
    )
    def sck(src_hbm, dst_hbm, tab_hbm, e_hbm, out_hbm,
            src_v, srcadj_v, dst_v, rows_v, m_v, zb_v, agg_sh, sem):
        cid = lax.axis_index("c")
        sid = lax.axis_index("s")

        def zrow(r, carry):
            for j in range(D // 16):
                zb_v[r, pl.ds(j * 16, 16)] = jnp.zeros((16,), jnp.float32)
            return carry

        lax.fori_loop(0, _RB, zrow, 0)
        for b in range(_RPT // _RB):
            pltpu.sync_copy(zb_v, agg_sh.at[pl.ds(sid * _RPT + b * _RB, _RB)])
        plsc.subcore_barrier()

        if feat_split:
            row_off = cid * N
            src_off = 0
            e_off = cid * E
        else:
            row_off = 0
            src_off = cid * (E // 2)
            e_off = cid * (E // 2)
        nk = kbase + jnp.where(sid < kextra, 1, 0)

        def chunk(k, carry):
            base = (k * 16 + sid) * _C
            pltpu.sync_copy(src_hbm.at[pl.ds(src_off + base, _C)], src_v)
            pltpu.sync_copy(dst_hbm.at[pl.ds(src_off + base, _C)], dst_v)
            for j in range(_C // 16):
                s = pl.ds(j * 16, 16)
                srcadj_v[s] = src_v[s] + row_off
            pltpu.async_copy(tab_hbm.at[srcadj_v], rows_v, sem).wait()
            pltpu.sync_copy(e_hbm.at[pl.ds(e_off + base, _C)], m_v)

            def mrow(r, c2):
                for j in range(D // 16):
                    s = pl.ds(j * 16, 16)
                    m_v[r, s] = jnp.maximum(m_v[r, s] + rows_v[r, s], 0.0)
                return c2

            lax.fori_loop(0, _C, mrow, 0)
            pltpu.sync_copy(m_v, agg_sh.at[dst_v], add=True)
            return carry

        lax.fori_loop(0, nk, chunk, 0)
        plsc.subcore_barrier()

        for b in range(_RPT // _RB):
            r0 = sid * _RPT + b * _RB
            pltpu.sync_copy(agg_sh.at[pl.ds(r0, _RB)], zb_v)
            pltpu.sync_copy(zb_v, out_hbm.at[pl.ds(cid * _NP + r0, _RB)])

    return sck


_make_sc_scatter = functools.lru_cache(maxsize=None)(_make_sc_scatter)


_DBG_EDGE_EMUL = False
_DBG_FEAT_EMUL = False


def _sc_scatter_edge(src, dst, tab, e):
    if _DBG_EDGE_EMUL:
        outs = []
        for c in range(2):
            sl = slice(c * E // 2, (c + 1) * E // 2)
            m = jnp.maximum(tab[src[sl]] + e[sl], 0.0)
            outs.append(jnp.zeros((_NP, 128), jnp.float32).at[dst[sl]].add(m))
        return jnp.concatenate(outs, axis=0)
    return _make_sc_scatter(False)(src, dst, tab, e)


def _sc_scatter_feat(src, dst, tab, e):
    if _DBG_FEAT_EMUL:
        outs = []
        for c in range(2):
            xr = tab[c * N:(c + 1) * N]
            er = e[c * E:(c + 1) * E]
            m = jnp.maximum(xr[src] + er, 0.0)
            outs.append(jnp.zeros((_NP, 128), jnp.float32).at[dst].add(m))
        return jnp.concatenate(outs, axis=0)
    return _make_sc_scatter(True)(src, dst, tab, e)


_NB = 2000


def _node1_body(x_ref, agg_ref, w1_ref, b1_ref, w2_ref, b2_ref, out_ref):
    h = x_ref[...] + agg_ref[0] + agg_ref[1]
    u = jnp.maximum(
        jnp.dot(h, w1_ref[...], preferred_element_type=jnp.float32) + b1_ref[...], 0.0)
    v = jnp.dot(u, w2_ref[...], preferred_element_type=jnp.float32) + b2_ref[...]
    h1 = jnp.maximum(v, 0.0)
    out_ref[0] = h1[:, :128]
    out_ref[1] = h1[:, 128:]


def _node1(x, agg1, w1, b1, w2, b2):
    nblk = N // _NB
    return pl.pallas_call(
        _node1_body,
        grid=(nblk,),
        in_specs=[
            pl.BlockSpec((_NB, D_IN), lambda i: (i, 0)),
            pl.BlockSpec((2, _NB, D_IN), lambda i: (0, i, 0)),
            pl.BlockSpec((D_IN, H), lambda i: (0, 0)),
            pl.BlockSpec((1, H), lambda i: (0, 0)),
            pl.BlockSpec((H, H), lambda i: (0, 0)),
            pl.BlockSpec((1, H), lambda i: (0, 0)),
        ],
        out_specs=pl.BlockSpec((2, _NB, 128), lambda i: (0, i, 0)),
        out_shape=jax.ShapeDtypeStruct((2, N, 128), jnp.float32),
    )(x, agg1, w1, b1, w2, b2)


def _node2_body(h1_ref, agg_ref, batch_ref, w1_ref, b1_ref, w2_ref, b2_ref,
                l1w_ref, l1b_ref, l2w_ref, l2b_ref, hw_ref, hb_ref,
                out_ref, sums_ref, cnts_ref):
    i = pl.program_id(0)

    @pl.when(i == 0)
    def _init():
        sums_ref[...] = jnp.zeros_like(sums_ref)
        cnts_ref[...] = jnp.zeros_like(cnts_ref)

    h = (jnp.concatenate([h1_ref[0], h1_ref[1]], axis=1)
         + jnp.concatenate([agg_ref[0], agg_ref[1]], axis=1))
    u = jnp.maximum(
        jnp.dot(h, w1_ref[...], preferred_element_type=jnp.float32) + b1_ref[...], 0.0)
    h2 = jnp.maximum(
        jnp.dot(u, w2_ref[...], preferred_element_type=jnp.float32) + b2_ref[...], 0.0)

    b = batch_ref[0]
    gid = lax.broadcasted_iota(jnp.int32, (G, 1), 0)
    p = (b == gid).astype(jnp.float32)
    sums_ref[...] += jnp.dot(p, h2, preferred_element_type=jnp.float32,
                             precision=lax.Precision.HIGHEST)
    cnts_ref[...] += jnp.sum(p, axis=1, keepdims=True)

    @pl.when(i == pl.num_programs(0) - 1)
    def _final():
        cn = jnp.maximum(cnts_ref[:, 0:1], 1.0)
        g = sums_ref[...] / cn
        g1 = jnp.maximum(
            jnp.dot(g, l1w_ref[...], preferred_element_type=jnp.float32) + l1b_ref[...], 0.0)
        g2 = jnp.maximum(
            jnp.dot(g1, l2w_ref[...], preferred_element_type=jnp.float32) + l2b_ref[...], 0.0)
        out_ref[...] = jnp.dot(g2, hw_ref[...], preferred_element_type=jnp.float32) + hb_ref[...]


def _node2(h1s, agg2, batch3, w1, b1, w2, b2, l1w, l1b, l2w, l2b, hw, hb):
    nblk = N // _NB
    return pl.pallas_call(
        _node2_body,
        grid=(nblk,),
        in_specs=[
            pl.BlockSpec((2, _NB, 128), lambda i: (0, i, 0)),
            pl.BlockSpec((2, _NB, 128), lambda i: (0, i, 0)),
            pl.BlockSpec((1, 1, _NB), lambda i: (i, 0, 0)),
            pl.BlockSpec((H, H), lambda i: (0, 0)),
            pl.BlockSpec((1, H), lambda i: (0, 0)),
            pl.BlockSpec((H, H), lambda i: (0, 0)),
            pl.BlockSpec((1, H), lambda i: (0, 0)),
            pl.BlockSpec((H, 128), lambda i: (0, 0)),
            pl.BlockSpec((1, 128), lambda i: (0, 0)),
            pl.BlockSpec((128, 64), lambda i: (0, 0)),
            pl.BlockSpec((1, 64), lambda i: (0, 0)),
            pl.BlockSpec((64, 3), lambda i: (0, 0)),
            pl.BlockSpec((1, 3), lambda i: (0, 0)),
        ],
        out_specs=pl.BlockSpec((G, 3), lambda i: (0, 0)),
        out_shape=jax.ShapeDtypeStruct((G, 3), jnp.float32),
        scratch_shapes=[
            pltpu.VMEM((G, H), jnp.float32),
            pltpu.VMEM((G, 128), jnp.float32),
        ],
    )(h1s, agg2, batch3, w1, b1, w2, b2, l1w, l1b, l2w, l2b, hw, hb)


def kernel(x, edge_index, edge_attr, batch,
           em_w1, em_b1, em_w2, em_b2,
           c1_lw, c1_lb, c1_w1, c1_b1, c1_w2, c1_b2,
           c2_lw, c2_lb, c2_w1, c2_b1, c2_w2, c2_b2,
           l1_w, l1_b, l2_w, l2_b, hS_w, hS_b, hP_w, hP_b, hN_w, hN_b):
    src = edge_index[0]
    dst = edge_index[1]

    e1, e2 = _edge_features(edge_attr, em_w1, em_b1[None], em_w2, em_b2[None],
                            c1_lw, c1_lb[None], c2_lw, c2_lb[None])

    agg1 = _sc_scatter_edge(src, dst, x, e1)

    h1s = _node1(x, agg1.reshape(2, _NP, D_IN)[:, :N], c1_w1, c1_b1[None],
                 c1_w2, c1_b2[None])

    agg2 = _sc_scatter_feat(src, dst, h1s.reshape(2 * N, 128),
                            e2.reshape(2 * E, 128))

    hw = jnp.concatenate([hS_w, hP_w, hN_w], axis=1)
    hb = jnp.concatenate([hS_b, hP_b, hN_b]).reshape(1, 3)
    out = _node2(h1s, agg2.reshape(2, _NP, 128)[:, :N],
                 batch.reshape(N // _NB, 1, _NB),
                 c2_w1, c2_b1[None], c2_w2, c2_b2[None],
                 l1_w, l1_b[None], l2_w, l2_b[None], hw, hb)
    return out[:, 0], out[:, 1], out[:, 2]
